# Initial kernel scaffold; baseline (speedup 1.0000x reference)
import jax
import jax.numpy as jnp
from jax.experimental import pallas as pl


def kernel(x, router_indices, LRA_mask, W_down, W_up):
    raise NotImplementedError("write your pallas kernel here")


# speedup vs baseline: 8.7889x; 8.7889x over previous
import jax
import jax.numpy as jnp
from jax.experimental import pallas as pl
from jax.experimental.pallas import tpu as pltpu

RANK = 16
PADK = 128
TBLK = 1024


def _lra_block(x_ref, ri_ref, colkey_ref, wd_ref, wu_ref, o_ref):
    xb = x_ref[...]
    ri = ri_ref[...]
    mask = colkey_ref[...] == ri
    down = jnp.dot(xb.astype(jnp.bfloat16), wd_ref[...],
                   preferred_element_type=jnp.float32)
    down = jnp.where(mask, down, 0.0)
    delta = jnp.dot(down.astype(jnp.bfloat16), wu_ref[...],
                    preferred_element_type=jnp.float32)
    o_ref[...] = xb + delta


def kernel(x, router_indices, LRA_mask, W_down, W_up):
    ntok, dim = x.shape
    nkeys, rank, _ = W_down.shape

    wd = jnp.transpose(W_down[LRA_mask], (2, 0, 1)).reshape(dim, nkeys * rank)
    wd = jnp.pad(wd, ((0, 0), (0, PADK - nkeys * rank))).astype(jnp.bfloat16)
    wu = jnp.transpose(W_up[LRA_mask], (0, 2, 1)).reshape(nkeys * rank, dim)
    wu = jnp.pad(wu, ((0, PADK - nkeys * rank), (0, 0))).astype(jnp.bfloat16)
    colkey = jnp.pad(jnp.repeat(LRA_mask, rank), (0, PADK - nkeys * rank),
                     constant_values=-1).reshape(1, PADK)

    grid = (ntok // TBLK,)
    return pl.pallas_call(
        _lra_block,
        grid=grid,
        in_specs=[
            pl.BlockSpec((TBLK, dim), lambda i: (i, 0)),
            pl.BlockSpec((TBLK, 1), lambda i: (i, 0)),
            pl.BlockSpec((1, PADK), lambda i: (0, 0)),
            pl.BlockSpec((dim, PADK), lambda i: (0, 0)),
            pl.BlockSpec((PADK, dim), lambda i: (0, 0)),
        ],
        out_specs=pl.BlockSpec((TBLK, dim), lambda i: (i, 0)),
        out_shape=jax.ShapeDtypeStruct((ntok, dim), x.dtype),
        compiler_params=pltpu.CompilerParams(
            dimension_semantics=("arbitrary",),
        ),
    )(x, router_indices, colkey, wd, wu)
